# baseline (speedup 1.0000x reference)
import jax
import jax.numpy as jnp
from jax.experimental import pallas as pl
from jax.experimental.pallas import tpu as pltpu

_B = 16384
_K = 8192
_D = 64
_TB = 256
_BETA = 0.25


def _normalize_rows(x, eps=1e-12):
    norm = jnp.linalg.norm(x, axis=1, keepdims=True)
    return x / jnp.maximum(norm, eps)


def _vq_body(nh_ref, nw_ref, w_ref, h_ref, aux_ref,
             onehot_ref, zq_ref, idx_ref, loss_ref, perp_ref,
             counts_ref, lacc_ref):
    b = pl.program_id(0)
    nb = pl.num_programs(0)

    nh = nh_ref[...]
    d = jax.lax.dot_general(
        nh, nw_ref[...], (((1,), (1,)), ((), ())),
        preferred_element_type=jnp.float32)
    m = jnp.max(d, axis=1, keepdims=True)
    onehot_ref[...] = (d >= m).astype(jnp.float32)

    agg = jax.lax.dot_general(
        onehot_ref[...], aux_ref[...], (((1,), (0,)), ((), ())),
        preferred_element_type=jnp.float32)
    nmax = agg[:, 2:3]
    idxf = agg[:, 0:1] * 64.0 + agg[:, 1:2]
    idx_ref[...] = idxf[:, 0].astype(jnp.int32)

    @pl.when(jnp.max(nmax) > 1.5)
    def _tie_fallback():
        col = jax.lax.broadcasted_iota(jnp.int32, (_TB, _K), 1)
        sel = jnp.where(d >= m, col, -1)
        idxv = jnp.max(sel, axis=1, keepdims=True)
        onehot_ref[...] = (col == idxv).astype(jnp.float32)
        idx_ref[...] = idxv[:, 0]

    oh = onehot_ref[...]
    zq = jax.lax.dot_general(
        oh, w_ref[...], (((1,), (0,)), ((), ())),
        preferred_element_type=jnp.float32)
    h = h_ref[...]
    zq_ref[...] = h + (zq - h)

    @pl.when(b == 0)
    def _init():
        lacc_ref[...] = jnp.zeros_like(lacc_ref)
        counts_ref[...] = jnp.zeros_like(counts_ref)
        perp_ref[...] = jnp.zeros_like(perp_ref)

    ones8 = jnp.ones((8, _TB), jnp.float32)
    counts_ref[...] += jax.lax.dot_general(
        ones8, oh, (((1,), (0,)), ((), ())),
        preferred_element_type=jnp.float32)

    diff = zq - h
    lacc_ref[...] += jnp.sum(diff * diff).reshape(1, 1)
    loss_ref[...] = (1.0 + _BETA) * (1.0 / (_B * _D)) * lacc_ref[...]

    @pl.when(b == nb - 1)
    def _fin():
        p = counts_ref[0:1, :] * (1.0 / _B)
        ent = jnp.sum(p * jnp.log(p + 1e-10))
        perp_ref[...] = jnp.exp(-ent).reshape(1, 1)


def kernel(h_batch, W):
    nh = _normalize_rows(jax.lax.stop_gradient(h_batch))
    nw = _normalize_rows(W)
    col = jnp.arange(_K, dtype=jnp.int32)
    aux = jnp.stack([(col >> 6).astype(jnp.float32),
                     (col & 63).astype(jnp.float32),
                     jnp.ones((_K,), jnp.float32)], axis=1)
    grid = (_B // _TB,)
    onehot, zq, idx, loss, perp = pl.pallas_call(
        _vq_body,
        grid=grid,
        in_specs=[
            pl.BlockSpec((_TB, _D), lambda b: (b, 0)),
            pl.BlockSpec((_K, _D), lambda b: (0, 0)),
            pl.BlockSpec((_K, _D), lambda b: (0, 0)),
            pl.BlockSpec((_TB, _D), lambda b: (b, 0)),
            pl.BlockSpec((_K, 3), lambda b: (0, 0)),
        ],
        out_specs=[
            pl.BlockSpec((_TB, _K), lambda b: (b, 0)),
            pl.BlockSpec((_TB, _D), lambda b: (b, 0)),
            pl.BlockSpec((_TB,), lambda b: (b,)),
            pl.BlockSpec((1, 1), lambda b: (0, 0)),
            pl.BlockSpec((1, 1), lambda b: (0, 0)),
        ],
        out_shape=[
            jax.ShapeDtypeStruct((_B, _K), jnp.float32),
            jax.ShapeDtypeStruct((_B, _D), jnp.float32),
            jax.ShapeDtypeStruct((_B,), jnp.int32),
            jax.ShapeDtypeStruct((1, 1), jnp.float32),
            jax.ShapeDtypeStruct((1, 1), jnp.float32),
        ],
        scratch_shapes=[
            pltpu.VMEM((8, _K), jnp.float32),
            pltpu.VMEM((1, 1), jnp.float32),
        ],
        compiler_params=pltpu.CompilerParams(
            dimension_semantics=("arbitrary",),
        ),
    )(nh, nw, W, h_batch, aux)
    return (zq, loss[0, 0], perp[0, 0], onehot, idx)


# speedup vs baseline: 1.4159x; 1.4159x over previous
import jax
import jax.numpy as jnp
from jax.experimental import pallas as pl
from jax.experimental.pallas import tpu as pltpu

_B = 16384
_K = 8192
_D = 64
_TB = 256
_BETA = 0.25


def _normalize_rows(x, eps=1e-12):
    norm = jnp.linalg.norm(x, axis=1, keepdims=True)
    return x / jnp.maximum(norm, eps)


def _vq_body(nh_ref, nw_ref, rhs_ref, h_ref,
             onehot_ref, zq_ref, idx_ref, loss_ref, perp_ref,
             counts_ref, lacc_ref):
    b = pl.program_id(0)
    nb = pl.num_programs(0)

    d = jax.lax.dot_general(
        nh_ref[...], nw_ref[...], (((1,), (1,)), ((), ())),
        preferred_element_type=jnp.float32)
    m = jnp.max(d, axis=1, keepdims=True)
    oh = (d >= m).astype(jnp.float32)
    onehot_ref[...] = oh
    oh_bf = oh.astype(jnp.bfloat16)

    agg = jax.lax.dot_general(
        oh_bf, rhs_ref[...], (((1,), (0,)), ((), ())),
        preferred_element_type=jnp.float32)
    zq = agg[:, 0:_D]
    idxf = agg[:, _D:_D + 1] * 64.0 + agg[:, _D + 1:_D + 2]
    nmax = agg[:, _D + 2:_D + 3]
    h = h_ref[...]
    zq_ref[...] = h + (zq - h)
    idx_ref[...] = idxf[:, 0].astype(jnp.int32)

    @pl.when(b == 0)
    def _init():
        lacc_ref[...] = jnp.zeros_like(lacc_ref)
        counts_ref[...] = jnp.zeros_like(counts_ref)
        perp_ref[...] = jnp.zeros_like(perp_ref)

    ones8 = jnp.ones((8, _TB), jnp.bfloat16)
    counts_ref[...] += jax.lax.dot_general(
        ones8, oh_bf, (((1,), (0,)), ((), ())),
        preferred_element_type=jnp.float32)

    @pl.when(jnp.max(nmax) > 1.5)
    def _tie_fallback():
        col = jax.lax.broadcasted_iota(jnp.int32, (_TB, _K), 1)
        sel = jnp.where(d >= m, col, -1)
        idxv = jnp.max(sel, axis=1, keepdims=True)
        oh2f = (col == idxv).astype(jnp.float32)
        onehot_ref[...] = oh2f
        oh2 = oh2f.astype(jnp.bfloat16)
        agg2 = jax.lax.dot_general(
            oh2, rhs_ref[...], (((1,), (0,)), ((), ())),
            preferred_element_type=jnp.float32)
        zq2 = agg2[:, 0:_D]
        zq_ref[...] = h + (zq2 - h)
        idx_ref[...] = idxv[:, 0]
        counts_ref[...] += jax.lax.dot_general(
            ones8, oh2 - oh_bf, (((1,), (0,)), ((), ())),
            preferred_element_type=jnp.float32)

    diff = zq_ref[...] - h
    lacc_ref[...] += jnp.sum(diff * diff).reshape(1, 1)
    loss_ref[...] = (1.0 + _BETA) * (1.0 / (_B * _D)) * lacc_ref[...]

    @pl.when(b == nb - 1)
    def _fin():
        p = counts_ref[0:1, :] * (1.0 / _B)
        ent = jnp.sum(p * jnp.log(p + 1e-10))
        perp_ref[...] = jnp.exp(-ent).reshape(1, 1)


def kernel(h_batch, W):
    nh = _normalize_rows(jax.lax.stop_gradient(h_batch))
    nw = _normalize_rows(W)
    col = jnp.arange(_K, dtype=jnp.int32)
    rhs = jnp.concatenate([
        W,
        (col >> 6).astype(jnp.float32)[:, None],
        (col & 63).astype(jnp.float32)[:, None],
        jnp.ones((_K, 1), jnp.float32),
        jnp.zeros((_K, 128 - _D - 3), jnp.float32),
    ], axis=1)
    nh_bf = nh.astype(jnp.bfloat16)
    nw_bf = nw.astype(jnp.bfloat16)
    rhs_bf = rhs.astype(jnp.bfloat16)
    grid = (_B // _TB,)
    onehot, zq, idx, loss, perp = pl.pallas_call(
        _vq_body,
        grid=grid,
        in_specs=[
            pl.BlockSpec((_TB, _D), lambda b: (b, 0)),
            pl.BlockSpec((_K, _D), lambda b: (0, 0)),
            pl.BlockSpec((_K, 128), lambda b: (0, 0)),
            pl.BlockSpec((_TB, _D), lambda b: (b, 0)),
        ],
        out_specs=[
            pl.BlockSpec((_TB, _K), lambda b: (b, 0)),
            pl.BlockSpec((_TB, _D), lambda b: (b, 0)),
            pl.BlockSpec((_TB,), lambda b: (b,)),
            pl.BlockSpec((1, 1), lambda b: (0, 0)),
            pl.BlockSpec((1, 1), lambda b: (0, 0)),
        ],
        out_shape=[
            jax.ShapeDtypeStruct((_B, _K), jnp.float32),
            jax.ShapeDtypeStruct((_B, _D), jnp.float32),
            jax.ShapeDtypeStruct((_B,), jnp.int32),
            jax.ShapeDtypeStruct((1, 1), jnp.float32),
            jax.ShapeDtypeStruct((1, 1), jnp.float32),
        ],
        scratch_shapes=[
            pltpu.VMEM((8, _K), jnp.float32),
            pltpu.VMEM((1, 1), jnp.float32),
        ],
        compiler_params=pltpu.CompilerParams(
            dimension_semantics=("arbitrary",),
        ),
    )(nh_bf, nw_bf, rhs_bf, h_batch)
    return (zq, loss[0, 0], perp[0, 0], onehot, idx)
